# initial kernel scaffold (unmeasured)
import jax
import jax.numpy as jnp
from jax import lax
from jax.experimental import pallas as pl
from jax.experimental.pallas import tpu as pltpu

N_DEV = 4
SQ = 256
D = 1024
SKV = 4096
H = 8
DH = 128
SCALE = 0.08838834764831843


def kernel(x, Wq, Wo, K_ext, V_ext):
    x2 = x.reshape(SQ, D)
    K = K_ext.reshape(SKV, H, DH)
    V = V_ext.reshape(SKV, H, DH)

    def body(x_ref, wq_ref, wo_ref, k_ref, v_ref, out_ref,
             src_buf, recv_buf, send_sems, recv_sems):
        my = lax.axis_index("i")

        barrier_sem = pltpu.get_barrier_semaphore()
        for d in range(1, N_DEV):
            pl.semaphore_signal(
                barrier_sem, inc=1,
                device_id=((my + d) % N_DEV,),
                device_id_type=pl.DeviceIdType.MESH,
            )
        pl.semaphore_wait(barrier_sem, N_DEV - 1)

        q = jnp.dot(x_ref[...], wq_ref[...],
                    preferred_element_type=jnp.float32)
        q = q.reshape(SQ, H, DH)
        o_heads = []
        for h in range(H):
            qh = q[:, h, :]
            kh = k_ref[:, h, :]
            s = lax.dot_general(
                qh, kh, (((1,), (1,)), ((), ())),
                preferred_element_type=jnp.float32,
            ) * SCALE
            m = jnp.max(s, axis=1, keepdims=True)
            p = jnp.exp(s - m)
            l = jnp.sum(p, axis=1, keepdims=True)
            oh = jnp.dot(p, v_ref[:, h, :],
                         preferred_element_type=jnp.float32) / l
            o_heads.append(oh)
        o = jnp.concatenate(o_heads, axis=1)
        src_buf[...] = jnp.dot(o, wo_ref[...],
                               preferred_element_type=jnp.float32)

        rdmas = []
        for d in range(1, N_DEV):
            slot = (N_DEV - 1) - d
            rdma = pltpu.make_async_remote_copy(
                src_ref=src_buf,
                dst_ref=recv_buf.at[slot],
                send_sem=send_sems.at[d - 1],
                recv_sem=recv_sems.at[slot],
                device_id=((my + d) % N_DEV,),
                device_id_type=pl.DeviceIdType.MESH,
            )
            rdma.start()
            rdmas.append(rdma)
        for rdma in rdmas:
            rdma.wait()

        out_ref[...] = (src_buf[...] + recv_buf[0] + recv_buf[1]
                        + recv_buf[2])

    out = pl.pallas_call(
        body,
        out_shape=jax.ShapeDtypeStruct((SQ, D), jnp.float32),
        in_specs=[pl.BlockSpec(memory_space=pltpu.VMEM)] * 5,
        out_specs=pl.BlockSpec(memory_space=pltpu.VMEM),
        scratch_shapes=[
            pltpu.VMEM((SQ, D), jnp.float32),
            pltpu.VMEM((N_DEV - 1, SQ, D), jnp.float32),
            pltpu.SemaphoreType.DMA((N_DEV - 1,)),
            pltpu.SemaphoreType.DMA((N_DEV - 1,)),
        ],
        compiler_params=pltpu.CompilerParams(collective_id=0),
    )(x2, Wq, Wo, K, V)
    return out.reshape(1, SQ, D)


# baseline (device time: 63382 ns/iter reference)
import jax
import jax.numpy as jnp
from jax import lax
from jax.experimental import pallas as pl
from jax.experimental.pallas import tpu as pltpu

N_DEV = 4
SQ = 256
D = 1024
SKV = 4096
H = 8
DH = 128
SCALE = 0.08838834764831843


def kernel(x, Wq, Wo, K_ext, V_ext):
    x2 = x.reshape(SQ, D)
    K = K_ext.reshape(SKV, H, DH)
    V = V_ext.reshape(SKV, H, DH)

    def body(x_ref, wq_ref, wo_ref, k_hbm, v_hbm, out_ref,
             src_buf, recv_buf, o_buf, k_buf, v_buf,
             send_sems, recv_sems, kv_sems):
        my = lax.axis_index("i")

        barrier_sem = pltpu.get_barrier_semaphore()
        for d in range(1, N_DEV):
            pl.semaphore_signal(
                barrier_sem, inc=1,
                device_id=((my + d) % N_DEV,),
                device_id_type=pl.DeviceIdType.MESH,
            )
        pl.semaphore_wait(barrier_sem, N_DEV - 1)

        def kv_copy(h, slot):
            kc = pltpu.make_async_copy(
                k_hbm.at[:, h, :], k_buf.at[slot], kv_sems.at[slot, 0])
            vc = pltpu.make_async_copy(
                v_hbm.at[:, h, :], v_buf.at[slot], kv_sems.at[slot, 1])
            return kc, vc

        kc0, vc0 = kv_copy(0, 0)
        kc0.start()
        vc0.start()

        q = jnp.dot(x_ref[...], wq_ref[...],
                    preferred_element_type=jnp.float32)
        q = q.reshape(SQ, H, DH)

        for h in range(H):
            slot = h % 2
            if h + 1 < H:
                kcn, vcn = kv_copy(h + 1, (h + 1) % 2)
                kcn.start()
                vcn.start()
            kc, vc = kv_copy(h, slot)
            kc.wait()
            vc.wait()
            s = lax.dot_general(
                q[:, h, :], k_buf[slot],
                (((1,), (1,)), ((), ())),
                preferred_element_type=jnp.float32,
            ) * SCALE
            m = jnp.max(s, axis=1, keepdims=True)
            p = jnp.exp(s - m)
            l = jnp.sum(p, axis=1, keepdims=True)
            o_buf[:, h * DH:(h + 1) * DH] = jnp.dot(
                p, v_buf[slot], preferred_element_type=jnp.float32) / l

        src_buf[...] = jnp.dot(o_buf[...], wo_ref[...],
                               preferred_element_type=jnp.float32)

        rdmas = []
        for d in range(1, N_DEV):
            slot = (N_DEV - 1) - d
            rdma = pltpu.make_async_remote_copy(
                src_ref=src_buf,
                dst_ref=recv_buf.at[slot],
                send_sem=send_sems.at[d - 1],
                recv_sem=recv_sems.at[slot],
                device_id=((my + d) % N_DEV,),
                device_id_type=pl.DeviceIdType.MESH,
            )
            rdma.start()
            rdmas.append(rdma)
        for rdma in rdmas:
            rdma.wait()

        out_ref[...] = (src_buf[...] + recv_buf[0] + recv_buf[1]
                        + recv_buf[2])

    out = pl.pallas_call(
        body,
        out_shape=jax.ShapeDtypeStruct((SQ, D), jnp.float32),
        in_specs=[
            pl.BlockSpec(memory_space=pltpu.VMEM),
            pl.BlockSpec(memory_space=pltpu.VMEM),
            pl.BlockSpec(memory_space=pltpu.VMEM),
            pl.BlockSpec(memory_space=pltpu.MemorySpace.HBM),
            pl.BlockSpec(memory_space=pltpu.MemorySpace.HBM),
        ],
        out_specs=pl.BlockSpec(memory_space=pltpu.VMEM),
        scratch_shapes=[
            pltpu.VMEM((SQ, D), jnp.float32),
            pltpu.VMEM((N_DEV - 1, SQ, D), jnp.float32),
            pltpu.VMEM((SQ, D), jnp.float32),
            pltpu.VMEM((2, SKV, DH), jnp.float32),
            pltpu.VMEM((2, SKV, DH), jnp.float32),
            pltpu.SemaphoreType.DMA((N_DEV - 1,)),
            pltpu.SemaphoreType.DMA((N_DEV - 1,)),
            pltpu.SemaphoreType.DMA((2, 2)),
        ],
        compiler_params=pltpu.CompilerParams(
            collective_id=0,
            vmem_limit_bytes=100 * 1024 * 1024,
        ),
    )(x2, Wq, Wo, K, V)
    return out.reshape(1, SQ, D)


# device time: 62705 ns/iter; 1.0108x vs baseline; 1.0108x over previous
import jax
import jax.numpy as jnp
from jax import lax
from jax.experimental import pallas as pl
from jax.experimental.pallas import tpu as pltpu

N_DEV = 4
SQ = 256
D = 1024
SKV = 4096
H = 8
DH = 128
SCALE = 0.08838834764831843


def kernel(x, Wq, Wo, K_ext, V_ext):
    x2 = x.reshape(SQ, D).astype(jnp.bfloat16)
    Wq_bf = Wq.astype(jnp.bfloat16)
    Wo_bf = Wo.astype(jnp.bfloat16)
    K = K_ext.reshape(SKV, H, DH)
    V = V_ext.reshape(SKV, H, DH)

    def body(x_ref, wq_ref, wo_ref, k_hbm, v_hbm, out_ref,
             src_buf, recv_buf, o_buf, k_buf, v_buf,
             send_sems, recv_sems, kv_sems):
        my = lax.axis_index("i")

        barrier_sem = pltpu.get_barrier_semaphore()
        for d in range(1, N_DEV):
            pl.semaphore_signal(
                barrier_sem, inc=1,
                device_id=((my + d) % N_DEV,),
                device_id_type=pl.DeviceIdType.MESH,
            )
        pl.semaphore_wait(barrier_sem, N_DEV - 1)

        def kv_copy(h, slot):
            kc = pltpu.make_async_copy(
                k_hbm.at[:, h, :], k_buf.at[slot], kv_sems.at[slot, 0])
            vc = pltpu.make_async_copy(
                v_hbm.at[:, h, :], v_buf.at[slot], kv_sems.at[slot, 1])
            return kc, vc

        kc0, vc0 = kv_copy(0, 0)
        kc0.start()
        vc0.start()

        q = jnp.dot(x_ref[...], wq_ref[...],
                    preferred_element_type=jnp.float32)
        q = q.astype(jnp.bfloat16).reshape(SQ, H, DH)

        for h in range(H):
            slot = h % 2
            if h + 1 < H:
                kcn, vcn = kv_copy(h + 1, (h + 1) % 2)
                kcn.start()
                vcn.start()
            kc, vc = kv_copy(h, slot)
            kc.wait()
            vc.wait()
            s = lax.dot_general(
                q[:, h, :], k_buf[slot].astype(jnp.bfloat16),
                (((1,), (1,)), ((), ())),
                preferred_element_type=jnp.float32,
            ) * SCALE
            m = jnp.max(s, axis=1, keepdims=True)
            p = jnp.exp(s - m)
            l = jnp.sum(p, axis=1, keepdims=True)
            oh = jnp.dot(p.astype(jnp.bfloat16),
                         v_buf[slot].astype(jnp.bfloat16),
                         preferred_element_type=jnp.float32) / l
            o_buf[:, h * DH:(h + 1) * DH] = oh.astype(jnp.bfloat16)

        src_buf[...] = jnp.dot(o_buf[...], wo_ref[...],
                               preferred_element_type=jnp.float32)

        rdmas = []
        for d in range(1, N_DEV):
            slot = (N_DEV - 1) - d
            rdma = pltpu.make_async_remote_copy(
                src_ref=src_buf,
                dst_ref=recv_buf.at[slot],
                send_sem=send_sems.at[d - 1],
                recv_sem=recv_sems.at[slot],
                device_id=((my + d) % N_DEV,),
                device_id_type=pl.DeviceIdType.MESH,
            )
            rdma.start()
            rdmas.append(rdma)
        for rdma in rdmas:
            rdma.wait()

        out_ref[...] = (src_buf[...] + recv_buf[0] + recv_buf[1]
                        + recv_buf[2])

    out = pl.pallas_call(
        body,
        out_shape=jax.ShapeDtypeStruct((SQ, D), jnp.float32),
        in_specs=[
            pl.BlockSpec(memory_space=pltpu.VMEM),
            pl.BlockSpec(memory_space=pltpu.VMEM),
            pl.BlockSpec(memory_space=pltpu.VMEM),
            pl.BlockSpec(memory_space=pltpu.MemorySpace.HBM),
            pl.BlockSpec(memory_space=pltpu.MemorySpace.HBM),
        ],
        out_specs=pl.BlockSpec(memory_space=pltpu.VMEM),
        scratch_shapes=[
            pltpu.VMEM((SQ, D), jnp.float32),
            pltpu.VMEM((N_DEV - 1, SQ, D), jnp.float32),
            pltpu.VMEM((SQ, D), jnp.bfloat16),
            pltpu.VMEM((2, SKV, DH), jnp.float32),
            pltpu.VMEM((2, SKV, DH), jnp.float32),
            pltpu.SemaphoreType.DMA((N_DEV - 1,)),
            pltpu.SemaphoreType.DMA((N_DEV - 1,)),
            pltpu.SemaphoreType.DMA((2, 2)),
        ],
        compiler_params=pltpu.CompilerParams(
            collective_id=0,
            vmem_limit_bytes=100 * 1024 * 1024,
        ),
    )(x2, Wq_bf, Wo_bf, K, V)
    return out.reshape(1, SQ, D)


# device time: 38046 ns/iter; 1.6659x vs baseline; 1.6481x over previous
import jax
import jax.numpy as jnp
from jax import lax
from jax.experimental import pallas as pl
from jax.experimental.pallas import tpu as pltpu

N_DEV = 4
SQ = 256
D = 1024
SKV = 4096
H = 8
DH = 128
SCALE = 0.08838834764831843
BF = jnp.bfloat16

NB = 2
NSLOT = 4
RB = SQ // NB


def kernel(x, Wq, Wo, K_ext, V_ext):
    x2 = x.reshape(SQ, D)
    K = K_ext.reshape(SKV, H, DH)
    V = V_ext.reshape(SKV, H, DH)

    def body(x_ref, wq_ref, wo_ref, k_hbm, v_hbm, out_ref,
             src_buf, recv_buf, o_buf, kf32, vf32, k_bf, v_bf,
             send_sems, recv_sems, kv_sems):
        my = lax.axis_index("i")

        barrier_sem = pltpu.get_barrier_semaphore()
        for d in range(1, N_DEV):
            pl.semaphore_signal(
                barrier_sem, inc=1,
                device_id=((my + d) % N_DEV,),
                device_id_type=pl.DeviceIdType.MESH,
            )
        pl.semaphore_wait(barrier_sem, N_DEV - 1)

        def kv_copy(h, slot):
            kc = pltpu.make_async_copy(
                k_hbm.at[:, h, :], kf32.at[slot], kv_sems.at[slot, 0])
            vc = pltpu.make_async_copy(
                v_hbm.at[:, h, :], vf32.at[slot], kv_sems.at[slot, 1])
            return kc, vc

        for hh in range(NSLOT):
            kcp, vcp = kv_copy(hh, hh)
            kcp.start()
            vcp.start()

        q = jnp.dot(x_ref[...].astype(BF), wq_ref[...].astype(BF),
                    preferred_element_type=jnp.float32)
        q = (q * SCALE).astype(BF).reshape(SQ, H, DH)
        wo_b = wo_ref[...].astype(BF)
        v_bf[:, :, DH:] = jnp.ones((H, SKV, 128), BF)

        rdmas = []
        for b in range(NB):
            rows = pl.ds(b * RB, RB)
            for h in range(H):
                if b == 0:
                    slot = h % NSLOT
                    kc, vc = kv_copy(h, slot)
                    kc.wait()
                    vc.wait()
                    k_bf[h] = kf32[slot].astype(BF)
                    v_bf[h, :, :DH] = vf32[slot].astype(BF)
                    if h + NSLOT < H:
                        kcn, vcn = kv_copy(h + NSLOT, slot)
                        kcn.start()
                        vcn.start()
                s = lax.dot_general(
                    q[b * RB:(b + 1) * RB, h, :], k_bf[h],
                    (((1,), (1,)), ((), ())),
                    preferred_element_type=jnp.float32,
                )
                p = jnp.exp(s).astype(BF)
                ohl = jnp.dot(p, v_bf[h],
                              preferred_element_type=jnp.float32)
                oh = ohl[:, :DH] / ohl[:, DH:DH + 1]
                o_buf[:, h * DH:(h + 1) * DH] = oh.astype(BF)

            src_buf[rows, :] = jnp.dot(
                o_buf[...], wo_b,
                preferred_element_type=jnp.float32).astype(BF)

            block_rdmas = []
            for d in range(1, N_DEV):
                slot = (N_DEV - 1) - d
                rdma = pltpu.make_async_remote_copy(
                    src_ref=src_buf.at[rows, :],
                    dst_ref=recv_buf.at[slot, rows, :],
                    send_sem=send_sems.at[b, d - 1],
                    recv_sem=recv_sems.at[b, slot],
                    device_id=((my + d) % N_DEV,),
                    device_id_type=pl.DeviceIdType.MESH,
                )
                rdma.start()
                block_rdmas.append(rdma)
            rdmas.append(block_rdmas)

        for b in range(NB):
            for rdma in rdmas[b]:
                rdma.wait()
            rows = pl.ds(b * RB, RB)
            out_ref[rows, :] = (
                src_buf[rows, :].astype(jnp.float32)
                + recv_buf[0, rows, :].astype(jnp.float32)
                + recv_buf[1, rows, :].astype(jnp.float32)
                + recv_buf[2, rows, :].astype(jnp.float32)
            )

    out = pl.pallas_call(
        body,
        out_shape=jax.ShapeDtypeStruct((SQ, D), jnp.float32),
        in_specs=[
            pl.BlockSpec(memory_space=pltpu.VMEM),
            pl.BlockSpec(memory_space=pltpu.VMEM),
            pl.BlockSpec(memory_space=pltpu.VMEM),
            pl.BlockSpec(memory_space=pltpu.MemorySpace.HBM),
            pl.BlockSpec(memory_space=pltpu.MemorySpace.HBM),
        ],
        out_specs=pl.BlockSpec(memory_space=pltpu.VMEM),
        scratch_shapes=[
            pltpu.VMEM((SQ, D), BF),
            pltpu.VMEM((N_DEV - 1, SQ, D), BF),
            pltpu.VMEM((RB, D), BF),
            pltpu.VMEM((NSLOT, SKV, DH), jnp.float32),
            pltpu.VMEM((NSLOT, SKV, DH), jnp.float32),
            pltpu.VMEM((H, SKV, DH), BF),
            pltpu.VMEM((H, SKV, DH + 128), BF),
            pltpu.SemaphoreType.DMA((NB, N_DEV - 1)),
            pltpu.SemaphoreType.DMA((NB, N_DEV - 1)),
            pltpu.SemaphoreType.DMA((NSLOT, 2)),
        ],
        compiler_params=pltpu.CompilerParams(
            collective_id=0,
            vmem_limit_bytes=100 * 1024 * 1024,
        ),
    )(x2, Wq, Wo, K, V)
    return out.reshape(1, SQ, D)
